# baseline (speedup 1.0000x reference)
import functools
import jax
import jax.numpy as jnp
from jax import lax
from jax.experimental import pallas as pl
from jax.experimental.pallas import tpu as pltpu
from jax.experimental.pallas import tpu_sc as plsc

_NCLS = 6
_CH = 256
_QLEN = 50000
_TEMP = 0.2
_JQ = 2000
_NBLK = _QLEN // _JQ
_NW = 32
_PIXW = 1024
_FCH = 128


def _sc_keys_kernel(fea_hbm, pred_hbm, part_hbm, cnt_hbm,
                    fea_v, pred_v, cls_v, keys_v, cnt_v):
    cid = lax.axis_index("c")
    sid = lax.axis_index("s")
    wid = sid * 2 + cid
    base = wid * _PIXW
    b = base // 4096
    off = base % 4096
    for c in range(_NCLS):
        pltpu.sync_copy(pred_hbm.at[b, c, pl.ds(off, _PIXW)], pred_v.at[c])

    def cls_body(gi, carry):
        p = [pred_v[c, pl.ds(gi * 16, 16)] for c in range(_NCLS)]
        best = p[0]
        bidx = jnp.zeros((16,), jnp.int32)
        for c in range(1, _NCLS):
            gt = p[c] > best
            best = jnp.where(gt, p[c], best)
            bidx = jnp.where(gt, c, bidx)
        cls_v[pl.ds(gi * 16, 16)] = bidx
        return tuple(carry[c] + jnp.where(bidx == c, 1.0, 0.0)
                     for c in range(_NCLS))

    cnts = lax.fori_loop(0, _PIXW // 16, cls_body,
                         tuple(jnp.zeros((16,), jnp.float32)
                               for _ in range(_NCLS)))
    for c in range(_NCLS):
        cnt_v[c, :] = cnts[c]
        for gch in range(_CH // 16):
            keys_v[c, pl.ds(gch * 16, 16)] = jnp.zeros((16,), jnp.float32)

    for chunk in range(_PIXW // _FCH):
        pltpu.sync_copy(fea_hbm.at[pl.ds(base + chunk * _FCH, _FCH), :], fea_v)

        def grp_body(gi, _):
            cvec = cls_v[pl.ds(chunk * _FCH + gi * 16, 16)]
            for j in range(16):
                cp = cvec[j]
                row = gi * 16 + j
                for gch in range(_CH // 16):
                    x = fea_v[row, pl.ds(gch * 16, 16)]
                    plsc.addupdate(keys_v.at[cp, pl.ds(gch * 16, 16)], x)
            return 0

        lax.fori_loop(0, _FCH // 16, grp_body, 0)
    pltpu.sync_copy(keys_v, part_hbm.at[wid])
    pltpu.sync_copy(cnt_v, cnt_hbm.at[wid])


def _loss_kernel(q_ref, part_ref, cntp_ref, out_ref, acc_ref, fc_ref,
                 gM_ref, cnt_ref):
    k = pl.program_id(0)

    @pl.when(k == 0)
    def _():
        ksum = jnp.zeros((_NCLS, _CH), jnp.float32)
        cs = jnp.zeros((_NCLS, 16), jnp.float32)
        for w in range(_NW):
            ksum = ksum + part_ref[w]
            cs = cs + cntp_ref[w]
        cnt = jnp.sum(cs, axis=1, keepdims=True)
        mean = ksum / jnp.where(cnt > 0, cnt, 1.0)
        nrm = jnp.sqrt(jnp.sum(mean * mean, axis=1, keepdims=True))
        gM_ref[...] = mean / jnp.where(nrm > 0, nrm, 1.0) / _TEMP
        cnt_ref[...] = cnt

    g = [gM_ref[c:c + 1, :] for c in range(_NCLS)]
    rows = 8
    gb = [jnp.tile(gc, (rows, 1)) for gc in g]

    def body(i, carry):
        base = i * rows
        q = [q_ref[c, pl.ds(base, rows), :] for c in range(_NCLS)]
        s = (q[0] + q[1]) + (q[2] + q[3]) + (q[4] + q[5])
        out = []
        for c in range(_NCLS):
            a1 = gb[c] * q[c]
            a2 = gb[c] * (s - q[c])
            out.append(carry[c] + jnp.exp(a1) + jnp.exp(a2))
        return tuple(out)

    init = tuple(jnp.zeros((rows, _CH), jnp.float32) for _ in range(_NCLS))
    accs = jax.lax.fori_loop(0, _JQ // rows, body, init, unroll=4)
    for c in range(_NCLS):
        red = jnp.sum(accs[c], axis=0, keepdims=True)

        @pl.when(k == 0)
        def _():
            acc_ref[c:c + 1, :] = red
            fc_ref[c:c + 1, :] = g[c] * q_ref[c, 0:1, :]

        @pl.when(k > 0)
        def _():
            acc_ref[c:c + 1, :] = acc_ref[c:c + 1, :] + red

    @pl.when(k == pl.num_programs(0) - 1)
    def _():
        vals = jnp.log(acc_ref[...]) - fc_ref[...]
        ce = jnp.sum(vals, axis=1, keepdims=True) / _CH
        w = jnp.where(cnt_ref[...] > 0, ce, 0.0)
        out_ref[...] = jnp.sum(w, axis=0, keepdims=True)


def kernel(fea, pred, queues):
    bs = fea.shape[0]
    hw = fea.shape[2] * fea.shape[3]
    fea_flat = fea.transpose(0, 2, 3, 1).reshape(bs * hw, _CH)
    pred_r = pred.reshape(bs, _NCLS, hw)
    q_t = queues.transpose(0, 2, 1)

    mesh = plsc.VectorSubcoreMesh(core_axis_name="c", subcore_axis_name="s")
    sc_keys = functools.partial(
        pl.kernel, _sc_keys_kernel, mesh=mesh,
        out_type=[
            jax.ShapeDtypeStruct((_NW, _NCLS, _CH), jnp.float32),
            jax.ShapeDtypeStruct((_NW, _NCLS, 16), jnp.float32),
        ],
        scratch_types=[
            pltpu.VMEM((_FCH, _CH), jnp.float32),
            pltpu.VMEM((_NCLS, _PIXW), jnp.float32),
            pltpu.VMEM((_PIXW,), jnp.int32),
            pltpu.VMEM((_NCLS, _CH), jnp.float32),
            pltpu.VMEM((_NCLS, 16), jnp.float32),
        ],
    )()
    part, cntp = sc_keys(fea_flat, pred_r)

    loss = pl.pallas_call(
        _loss_kernel,
        grid=(_NBLK,),
        in_specs=[
            pl.BlockSpec((_NCLS, _JQ, _CH), lambda k: (0, k, 0)),
            pl.BlockSpec((_NW, _NCLS, _CH), lambda k: (0, 0, 0)),
            pl.BlockSpec((_NW, _NCLS, 16), lambda k: (0, 0, 0)),
        ],
        out_specs=pl.BlockSpec((1, 1), lambda k: (0, 0)),
        out_shape=jax.ShapeDtypeStruct((1, 1), jnp.float32),
        scratch_shapes=[
            pltpu.VMEM((_NCLS, _CH), jnp.float32),
            pltpu.VMEM((_NCLS, _CH), jnp.float32),
            pltpu.VMEM((_NCLS, _CH), jnp.float32),
            pltpu.VMEM((_NCLS, 1), jnp.float32),
        ],
    )(q_t, part, cntp)
    return loss[0, 0]


# speedup vs baseline: 1.0717x; 1.0717x over previous
import functools
import jax
import jax.numpy as jnp
from jax import lax
from jax.experimental import pallas as pl
from jax.experimental.pallas import tpu as pltpu
from jax.experimental.pallas import tpu_sc as plsc

_NCLS = 6
_CH = 256
_QLEN = 50000
_TEMP = 0.2
_JQ = 2000
_NBLK = _QLEN // _JQ
_NW = 32
_PIXW = 1024
_FCH = 128


def _sc_keys_kernel(fea_hbm, pred_hbm, part_hbm, cnt_hbm,
                    fea_v, pred_v, cls_v, keys_v, cnt_v, sem0, sem1):
    cid = lax.axis_index("c")
    sid = lax.axis_index("s")
    wid = sid * 2 + cid
    base = wid * _PIXW
    b = base // 4096
    off = base % 4096
    sems = (sem0, sem1)
    pltpu.async_copy(fea_hbm.at[pl.ds(base, _FCH), :], fea_v.at[0], sem0)
    for c in range(_NCLS):
        pltpu.sync_copy(pred_hbm.at[b, c, pl.ds(off, _PIXW)], pred_v.at[c])

    def cls_body(gi, carry):
        p = [pred_v[c, pl.ds(gi * 16, 16)] for c in range(_NCLS)]
        best = p[0]
        bidx = jnp.zeros((16,), jnp.int32)
        for c in range(1, _NCLS):
            gt = p[c] > best
            best = jnp.where(gt, p[c], best)
            bidx = jnp.where(gt, c, bidx)
        cls_v[pl.ds(gi * 16, 16)] = bidx
        return tuple(carry[c] + jnp.where(bidx == c, 1.0, 0.0)
                     for c in range(_NCLS))

    cnts = lax.fori_loop(0, _PIXW // 16, cls_body,
                         tuple(jnp.zeros((16,), jnp.float32)
                               for _ in range(_NCLS)))
    for c in range(_NCLS):
        cnt_v[c, :] = cnts[c]
        for gch in range(_CH // 16):
            keys_v[pl.ds(c * _CH + gch * 16, 16)] = jnp.zeros((16,),
                                                              jnp.float32)

    nchunk = _PIXW // _FCH
    for chunk in range(nchunk):
        pltpu.make_async_copy(fea_hbm.at[pl.ds(base + chunk * _FCH, _FCH), :],
                              fea_v.at[chunk % 2], sems[chunk % 2]).wait()
        if chunk + 1 < nchunk:
            pltpu.async_copy(
                fea_hbm.at[pl.ds(base + (chunk + 1) * _FCH, _FCH), :],
                fea_v.at[(chunk + 1) % 2], sems[(chunk + 1) % 2])

        def grp_body(gi, _):
            cvec = cls_v[pl.ds(chunk * _FCH + gi * 16, 16)]
            for j in range(16):
                cp = cvec[j]
                rowbase = cp * _CH
                row = gi * 16 + j
                for gch in range(_CH // 16):
                    x = fea_v[chunk % 2, row, pl.ds(gch * 16, 16)]
                    plsc.addupdate(
                        keys_v.at[pl.ds(rowbase + gch * 16, 16)], x)
            return 0

        lax.fori_loop(0, _FCH // 16, grp_body, 0)
    for c in range(_NCLS):
        pltpu.sync_copy(keys_v.at[pl.ds(c * _CH, _CH)], part_hbm.at[wid, c])
    pltpu.sync_copy(cnt_v, cnt_hbm.at[wid])


def _loss_kernel(q_ref, part_ref, cntp_ref, out_ref, acc_ref, fc_ref,
                 gM_ref, cnt_ref):
    k = pl.program_id(0)

    @pl.when(k == 0)
    def _():
        ksum = jnp.zeros((_NCLS, _CH), jnp.float32)
        cs = jnp.zeros((_NCLS, 16), jnp.float32)
        for w in range(_NW):
            ksum = ksum + part_ref[w]
            cs = cs + cntp_ref[w]
        cnt = jnp.sum(cs, axis=1, keepdims=True)
        mean = ksum / jnp.where(cnt > 0, cnt, 1.0)
        nrm = jnp.sqrt(jnp.sum(mean * mean, axis=1, keepdims=True))
        gM_ref[...] = mean / jnp.where(nrm > 0, nrm, 1.0) / _TEMP
        cnt_ref[...] = cnt

    g = [gM_ref[c:c + 1, :] for c in range(_NCLS)]
    rows = 8
    gb = [jnp.tile(gc, (rows, 1)) for gc in g]

    def body(i, carry):
        base = i * rows
        q = [q_ref[c, pl.ds(base, rows), :] for c in range(_NCLS)]
        s = (q[0] + q[1]) + (q[2] + q[3]) + (q[4] + q[5])
        out = []
        for c in range(_NCLS):
            a1 = gb[c] * q[c]
            a2 = gb[c] * (s - q[c])
            out.append(carry[c] + jnp.exp(a1) + jnp.exp(a2))
        return tuple(out)

    init = tuple(jnp.zeros((rows, _CH), jnp.float32) for _ in range(_NCLS))
    accs = jax.lax.fori_loop(0, _JQ // rows, body, init, unroll=4)
    for c in range(_NCLS):
        red = jnp.sum(accs[c], axis=0, keepdims=True)

        @pl.when(k == 0)
        def _():
            acc_ref[c:c + 1, :] = red
            fc_ref[c:c + 1, :] = g[c] * q_ref[c, 0:1, :]

        @pl.when(k > 0)
        def _():
            acc_ref[c:c + 1, :] = acc_ref[c:c + 1, :] + red

    @pl.when(k == pl.num_programs(0) - 1)
    def _():
        vals = jnp.log(acc_ref[...]) - fc_ref[...]
        ce = jnp.sum(vals, axis=1, keepdims=True) / _CH
        w = jnp.where(cnt_ref[...] > 0, ce, 0.0)
        out_ref[...] = jnp.sum(w, axis=0, keepdims=True)


def kernel(fea, pred, queues):
    bs = fea.shape[0]
    hw = fea.shape[2] * fea.shape[3]
    fea_flat = fea.transpose(0, 2, 3, 1).reshape(bs * hw, _CH)
    pred_r = pred.reshape(bs, _NCLS, hw)
    q_t = queues.transpose(0, 2, 1)

    mesh = plsc.VectorSubcoreMesh(core_axis_name="c", subcore_axis_name="s")
    sc_keys = functools.partial(
        pl.kernel, _sc_keys_kernel, mesh=mesh,
        out_type=[
            jax.ShapeDtypeStruct((_NW, _NCLS, _CH), jnp.float32),
            jax.ShapeDtypeStruct((_NW, _NCLS, 16), jnp.float32),
        ],
        scratch_types=[
            pltpu.VMEM((2, _FCH, _CH), jnp.float32),
            pltpu.VMEM((_NCLS, _PIXW), jnp.float32),
            pltpu.VMEM((_PIXW,), jnp.int32),
            pltpu.VMEM((_NCLS * _CH,), jnp.float32),
            pltpu.VMEM((_NCLS, 16), jnp.float32),
            pltpu.SemaphoreType.DMA,
            pltpu.SemaphoreType.DMA,
        ],
    )()
    part, cntp = sc_keys(fea_flat, pred_r)

    loss = pl.pallas_call(
        _loss_kernel,
        grid=(_NBLK,),
        in_specs=[
            pl.BlockSpec((_NCLS, _JQ, _CH), lambda k: (0, k, 0)),
            pl.BlockSpec((_NW, _NCLS, _CH), lambda k: (0, 0, 0)),
            pl.BlockSpec((_NW, _NCLS, 16), lambda k: (0, 0, 0)),
        ],
        out_specs=pl.BlockSpec((1, 1), lambda k: (0, 0)),
        out_shape=jax.ShapeDtypeStruct((1, 1), jnp.float32),
        scratch_shapes=[
            pltpu.VMEM((_NCLS, _CH), jnp.float32),
            pltpu.VMEM((_NCLS, _CH), jnp.float32),
            pltpu.VMEM((_NCLS, _CH), jnp.float32),
            pltpu.VMEM((_NCLS, 1), jnp.float32),
        ],
    )(q_t, part, cntp)
    return loss[0, 0]


# speedup vs baseline: 1.3727x; 1.2809x over previous
import jax
import jax.numpy as jnp
from jax import lax
from jax.experimental import pallas as pl
from jax.experimental.pallas import tpu as pltpu
from jax.experimental.pallas import tpu_sc as plsc

_NCLS = 6
_CH = 256
_QLEN = 50000
_TEMP = 0.2
_JQ = 2000
_NBLK = _QLEN // _JQ
_NW = 32
_PIXW = 1024


def _sc_route_kernel(pred_hbm, cls_hbm, cnt_hbm, pred_v, cls_v, cnt_v):
    cid = lax.axis_index("c")
    sid = lax.axis_index("s")
    wid = sid * 2 + cid
    base = wid * _PIXW
    b = base // 4096
    off = base % 4096
    for c in range(_NCLS):
        pltpu.sync_copy(pred_hbm.at[b, c, pl.ds(off, _PIXW)], pred_v.at[c])

    def cls_body(gi, carry):
        p = [pred_v[c, pl.ds(gi * 16, 16)] for c in range(_NCLS)]
        best = p[0]
        bidx = jnp.zeros((16,), jnp.int32)
        for c in range(1, _NCLS):
            gt = p[c] > best
            best = jnp.where(gt, p[c], best)
            bidx = jnp.where(gt, c, bidx)
        cls_v[pl.ds(gi * 16, 16)] = bidx
        return tuple(carry[c] + jnp.where(bidx == c, 1.0, 0.0)
                     for c in range(_NCLS))

    cnts = lax.fori_loop(0, _PIXW // 16, cls_body,
                         tuple(jnp.zeros((16,), jnp.float32)
                               for _ in range(_NCLS)))
    for c in range(_NCLS):
        cnt_v[c, :] = cnts[c]
    pltpu.sync_copy(cls_v, cls_hbm.at[wid])
    pltpu.sync_copy(cnt_v, cnt_hbm.at[wid])


def _keys_kernel(fea_ref, cls_ref, cntp_ref, gM_ref, cnt_ref):
    b = pl.program_id(0)
    fea = fea_ref[0]
    row = cls_ref[pl.ds(b, 1), :]
    cls = jax.lax.broadcasted_iota(jnp.int32, (_NCLS, row.shape[1]), 0)
    masks = (cls == row).astype(jnp.float32)
    keys_part = jax.lax.dot_general(
        masks, fea, (((1,), (0,)), ((), ())),
        preferred_element_type=jnp.float32,
        precision=jax.lax.Precision.HIGHEST)

    @pl.when(b == 0)
    def _():
        gM_ref[...] = keys_part
        cs = jnp.zeros((_NCLS, 16), jnp.float32)
        for w in range(_NW):
            cs = cs + cntp_ref[w]
        cnt_ref[...] = jnp.sum(cs, axis=1, keepdims=True)

    @pl.when(b > 0)
    def _():
        gM_ref[...] = gM_ref[...] + keys_part

    @pl.when(b == pl.num_programs(0) - 1)
    def _():
        ksum = gM_ref[...]
        cnt = cnt_ref[...]
        mean = ksum / jnp.where(cnt > 0, cnt, 1.0)
        nrm = jnp.sqrt(jnp.sum(mean * mean, axis=1, keepdims=True))
        gM_ref[...] = mean / jnp.where(nrm > 0, nrm, 1.0) / _TEMP


def _loss_kernel(q_ref, gM_ref, cnt_ref, out_ref, acc_ref, fc_ref):
    k = pl.program_id(0)
    g = [gM_ref[c:c + 1, :] for c in range(_NCLS)]
    rows = 8
    gb = [jnp.tile(gc, (rows, 1)) for gc in g]

    def body(i, carry):
        base = i * rows
        q = [q_ref[c, pl.ds(base, rows), :] for c in range(_NCLS)]
        s = (q[0] + q[1]) + (q[2] + q[3]) + (q[4] + q[5])
        out = []
        for c in range(_NCLS):
            a1 = gb[c] * q[c]
            a2 = gb[c] * (s - q[c])
            out.append(carry[c] + jnp.exp(a1) + jnp.exp(a2))
        return tuple(out)

    init = tuple(jnp.zeros((rows, _CH), jnp.float32) for _ in range(_NCLS))
    accs = jax.lax.fori_loop(0, _JQ // rows, body, init, unroll=4)
    for c in range(_NCLS):
        red = jnp.sum(accs[c], axis=0, keepdims=True)

        @pl.when(k == 0)
        def _():
            acc_ref[c:c + 1, :] = red
            fc_ref[c:c + 1, :] = g[c] * q_ref[c, 0:1, :]

        @pl.when(k > 0)
        def _():
            acc_ref[c:c + 1, :] = acc_ref[c:c + 1, :] + red

    @pl.when(k == pl.num_programs(0) - 1)
    def _():
        vals = jnp.log(acc_ref[...]) - fc_ref[...]
        ce = jnp.sum(vals, axis=1, keepdims=True) / _CH
        w = jnp.where(cnt_ref[...] > 0, ce, 0.0)
        out_ref[...] = jnp.sum(w, axis=0, keepdims=True)


def kernel(fea, pred, queues):
    bs = fea.shape[0]
    hw = fea.shape[2] * fea.shape[3]
    fea_t = fea.transpose(0, 2, 3, 1).reshape(bs, hw, _CH)
    pred_r = pred.reshape(bs, _NCLS, hw)
    q_t = queues.transpose(0, 2, 1)

    mesh = plsc.VectorSubcoreMesh(core_axis_name="c", subcore_axis_name="s")
    sc_route = pl.kernel(
        _sc_route_kernel, mesh=mesh,
        out_type=[
            jax.ShapeDtypeStruct((_NW, _PIXW), jnp.int32),
            jax.ShapeDtypeStruct((_NW, _NCLS, 16), jnp.float32),
        ],
        scratch_types=[
            pltpu.VMEM((_NCLS, _PIXW), jnp.float32),
            pltpu.VMEM((_PIXW,), jnp.int32),
            pltpu.VMEM((_NCLS, 16), jnp.float32),
        ],
    )
    cls, cntp = sc_route(pred_r)
    cls_r = cls.reshape(bs, hw)

    gM, cnt = pl.pallas_call(
        _keys_kernel,
        grid=(bs,),
        in_specs=[
            pl.BlockSpec((1, hw, _CH), lambda b: (b, 0, 0)),
            pl.BlockSpec((bs, hw), lambda b: (0, 0)),
            pl.BlockSpec((_NW, _NCLS, 16), lambda b: (0, 0, 0)),
        ],
        out_specs=[
            pl.BlockSpec((_NCLS, _CH), lambda b: (0, 0)),
            pl.BlockSpec((_NCLS, 1), lambda b: (0, 0)),
        ],
        out_shape=[
            jax.ShapeDtypeStruct((_NCLS, _CH), jnp.float32),
            jax.ShapeDtypeStruct((_NCLS, 1), jnp.float32),
        ],
    )(fea_t, cls_r, cntp)

    loss = pl.pallas_call(
        _loss_kernel,
        grid=(_NBLK,),
        in_specs=[
            pl.BlockSpec((_NCLS, _JQ, _CH), lambda k: (0, k, 0)),
            pl.BlockSpec((_NCLS, _CH), lambda k: (0, 0)),
            pl.BlockSpec((_NCLS, 1), lambda k: (0, 0)),
        ],
        out_specs=pl.BlockSpec((1, 1), lambda k: (0, 0)),
        out_shape=jax.ShapeDtypeStruct((1, 1), jnp.float32),
        scratch_shapes=[
            pltpu.VMEM((_NCLS, _CH), jnp.float32),
            pltpu.VMEM((_NCLS, _CH), jnp.float32),
        ],
    )(q_t, gM, cnt)
    return loss[0, 0]


# speedup vs baseline: 1.3949x; 1.0161x over previous
import jax
import jax.numpy as jnp
from jax import lax
from jax.experimental import pallas as pl
from jax.experimental.pallas import tpu as pltpu
from jax.experimental.pallas import tpu_sc as plsc

_NCLS = 6
_CH = 256
_QLEN = 50000
_TEMP = 0.2
_JQ = 2000
_NBLK = _QLEN // _JQ
_NW = 32
_PIXW = 1024


def _sc_route_kernel(pred_hbm, cls_hbm, cnt_hbm, pred_v, cls_v, cnt_v):
    cid = lax.axis_index("c")
    sid = lax.axis_index("s")
    wid = sid * 2 + cid
    base = wid * _PIXW
    b = base // 4096
    off = base % 4096
    pltpu.sync_copy(pred_hbm.at[b, :, pl.ds(off, _PIXW)], pred_v)

    def cls_body(gi, carry):
        p = [pred_v[c, pl.ds(gi * 16, 16)] for c in range(_NCLS)]
        best = p[0]
        bidx = jnp.zeros((16,), jnp.int32)
        for c in range(1, _NCLS):
            gt = p[c] > best
            best = jnp.where(gt, p[c], best)
            bidx = jnp.where(gt, c, bidx)
        cls_v[pl.ds(gi * 16, 16)] = bidx
        return tuple(carry[c] + jnp.where(bidx == c, 1.0, 0.0)
                     for c in range(_NCLS))

    cnts = lax.fori_loop(0, _PIXW // 16, cls_body,
                         tuple(jnp.zeros((16,), jnp.float32)
                               for _ in range(_NCLS)))
    for c in range(_NCLS):
        cnt_v[c, :] = cnts[c]
    pltpu.sync_copy(cls_v, cls_hbm.at[wid])
    pltpu.sync_copy(cnt_v, cnt_hbm.at[wid])


def _keys_kernel(fea_ref, cls_ref, cntp_ref, gM_ref, cnt_ref):
    b = pl.program_id(0)
    fea = fea_ref[0]
    row = cls_ref[pl.ds(b, 1), :]
    cls = jax.lax.broadcasted_iota(jnp.int32, (_NCLS, row.shape[1]), 0)
    masks = (cls == row).astype(jnp.float32)
    keys_part = jax.lax.dot_general(
        masks, fea, (((1,), (0,)), ((), ())),
        preferred_element_type=jnp.float32,
        precision=jax.lax.Precision.HIGHEST)

    @pl.when(b == 0)
    def _():
        gM_ref[...] = keys_part
        cs = jnp.zeros((_NCLS, 16), jnp.float32)
        for w in range(_NW):
            cs = cs + cntp_ref[w]
        cnt_ref[...] = jnp.sum(cs, axis=1, keepdims=True)

    @pl.when(b > 0)
    def _():
        gM_ref[...] = gM_ref[...] + keys_part

    @pl.when(b == pl.num_programs(0) - 1)
    def _():
        ksum = gM_ref[...]
        cnt = cnt_ref[...]
        mean = ksum / jnp.where(cnt > 0, cnt, 1.0)
        nrm = jnp.sqrt(jnp.sum(mean * mean, axis=1, keepdims=True))
        gM_ref[...] = mean / jnp.where(nrm > 0, nrm, 1.0) / _TEMP


def _loss_kernel(q_ref, gM_ref, cnt_ref, out_ref, acc_ref, fc_ref):
    k = pl.program_id(0)
    g = [gM_ref[c:c + 1, :] for c in range(_NCLS)]
    rows = 8
    gb = [jnp.tile(gc, (rows, 1)) for gc in g]

    def body(i, carry):
        base = i * rows
        q = [q_ref[c, pl.ds(base, rows), :] for c in range(_NCLS)]
        s = (q[0] + q[1]) + (q[2] + q[3]) + (q[4] + q[5])
        out = []
        for c in range(_NCLS):
            a1 = gb[c] * q[c]
            a2 = gb[c] * (s - q[c])
            out.append(carry[c] + jnp.exp(a1) + jnp.exp(a2))
        return tuple(out)

    init = tuple(jnp.zeros((rows, _CH), jnp.float32) for _ in range(_NCLS))
    accs = jax.lax.fori_loop(0, _JQ // rows, body, init, unroll=4)
    for c in range(_NCLS):
        red = jnp.sum(accs[c], axis=0, keepdims=True)

        @pl.when(k == 0)
        def _():
            acc_ref[c:c + 1, :] = red
            fc_ref[c:c + 1, :] = g[c] * q_ref[c, 0:1, :]

        @pl.when(k > 0)
        def _():
            acc_ref[c:c + 1, :] = acc_ref[c:c + 1, :] + red

    @pl.when(k == pl.num_programs(0) - 1)
    def _():
        vals = jnp.log(acc_ref[...]) - fc_ref[...]
        ce = jnp.sum(vals, axis=1, keepdims=True) / _CH
        w = jnp.where(cnt_ref[...] > 0, ce, 0.0)
        out_ref[...] = jnp.sum(w, axis=0, keepdims=True)


def kernel(fea, pred, queues):
    bs = fea.shape[0]
    hw = fea.shape[2] * fea.shape[3]
    fea_t = fea.transpose(0, 2, 3, 1).reshape(bs, hw, _CH)
    pred_r = pred.reshape(bs, _NCLS, hw)
    q_t = queues.transpose(0, 2, 1)

    mesh = plsc.VectorSubcoreMesh(core_axis_name="c", subcore_axis_name="s")
    sc_route = pl.kernel(
        _sc_route_kernel, mesh=mesh,
        out_type=[
            jax.ShapeDtypeStruct((_NW, _PIXW), jnp.int32),
            jax.ShapeDtypeStruct((_NW, _NCLS, 16), jnp.float32),
        ],
        scratch_types=[
            pltpu.VMEM((_NCLS, _PIXW), jnp.float32),
            pltpu.VMEM((_PIXW,), jnp.int32),
            pltpu.VMEM((_NCLS, 16), jnp.float32),
        ],
    )
    cls, cntp = sc_route(pred_r)
    cls_r = cls.reshape(bs, hw)

    gM, cnt = pl.pallas_call(
        _keys_kernel,
        grid=(bs,),
        in_specs=[
            pl.BlockSpec((1, hw, _CH), lambda b: (b, 0, 0)),
            pl.BlockSpec((bs, hw), lambda b: (0, 0)),
            pl.BlockSpec((_NW, _NCLS, 16), lambda b: (0, 0, 0)),
        ],
        out_specs=[
            pl.BlockSpec((_NCLS, _CH), lambda b: (0, 0)),
            pl.BlockSpec((_NCLS, 1), lambda b: (0, 0)),
        ],
        out_shape=[
            jax.ShapeDtypeStruct((_NCLS, _CH), jnp.float32),
            jax.ShapeDtypeStruct((_NCLS, 1), jnp.float32),
        ],
    )(fea_t, cls_r, cntp)

    loss = pl.pallas_call(
        _loss_kernel,
        grid=(_NBLK,),
        in_specs=[
            pl.BlockSpec((_NCLS, _JQ, _CH), lambda k: (0, k, 0)),
            pl.BlockSpec((_NCLS, _CH), lambda k: (0, 0)),
            pl.BlockSpec((_NCLS, 1), lambda k: (0, 0)),
        ],
        out_specs=pl.BlockSpec((1, 1), lambda k: (0, 0)),
        out_shape=jax.ShapeDtypeStruct((1, 1), jnp.float32),
        scratch_shapes=[
            pltpu.VMEM((_NCLS, _CH), jnp.float32),
            pltpu.VMEM((_NCLS, _CH), jnp.float32),
        ],
    )(q_t, gM, cnt)
    return loss[0, 0]
